# Initial kernel scaffold; baseline (speedup 1.0000x reference)
import jax
import jax.numpy as jnp
from jax.experimental import pallas as pl


def kernel(x, edge_index, batch, Wrel0, Wroot0, b0, Wrel1, Wroot1, b1, Wrel2, Wroot2, b2, lin1_W, lin1_b, lin2_W, lin2_b):
    raise NotImplementedError("write your pallas kernel here")


# speedup vs baseline: 2.6752x; 2.6752x over previous
import functools

import jax
import jax.numpy as jnp
from jax import lax
from jax.experimental import pallas as pl
from jax.experimental.pallas import tpu as pltpu
from jax.experimental.pallas import tpu_sc as plsc

N, E, D, H, G = 10000, 320000, 128, 256, 64
NC, NS = 2, 16
CH = 256
EPAD = 327680
CPT0 = EPAD // 2 // NS // CH
CPT12 = EPAD // NS // CH
OPT = 632
NOUT = NS * OPT
NACC = NOUT
ZPT = NACC // NS
RB = 1000
G2 = 128

_mesh = plsc.VectorSubcoreMesh(core_axis_name="c", subcore_axis_name="s")


def _sc_segsum(cpt):
    @functools.partial(
        pl.kernel,
        mesh=_mesh,
        out_type=jax.ShapeDtypeStruct((NC, NOUT, D), jnp.float32),
        scratch_types=[
            pltpu.VMEM((CH,), jnp.int32),
            pltpu.VMEM((CH,), jnp.int32),
            pltpu.VMEM((CH, D), jnp.float32),
            pltpu.VMEM((128, D), jnp.float32),
            pltpu.VMEM_SHARED((NACC, D), jnp.float32),
            pltpu.SemaphoreType.DMA,
        ],
    )
    def seg(table, si, di, zr, out, idx_v, didx_v, rows_v, zv, acc, sem):
        c = lax.axis_index("c")
        s = lax.axis_index("s")

        pltpu.sync_copy(zr, zv)
        base = s * ZPT
        for k in range(ZPT // 128):
            pltpu.sync_copy(zv, acc.at[pl.ds(base + k * 128, 128)])
        rem = ZPT % 128
        pltpu.sync_copy(zv.at[pl.ds(0, rem)],
                        acc.at[pl.ds(base + ZPT - rem, rem)])
        plsc.subcore_barrier()

        def chunk(j, carry):
            pltpu.sync_copy(si.at[c, s, j], idx_v)
            pltpu.sync_copy(di.at[c, s, j], didx_v)
            pltpu.async_copy(table.at[idx_v], rows_v, sem).wait()
            pltpu.sync_copy(rows_v, acc.at[didx_v], add=True)
            return carry

        lax.fori_loop(0, cpt, chunk, 0)
        plsc.subcore_barrier()

        pltpu.sync_copy(acc.at[pl.ds(s * OPT, OPT)],
                        out.at[c, pl.ds(s * OPT, OPT)])

    return seg


def _tc_layer(Ka, Kh):
    def body(a_ref, wa_ref, h_ref, wh_ref, b_ref, o_ref):
        acc = jnp.zeros((RB, D), jnp.float32) + b_ref[...]
        for k in range(Ka):
            acc = acc + jnp.dot(a_ref[k], wa_ref[k],
                                preferred_element_type=jnp.float32)
        for k in range(Kh):
            acc = acc + jnp.dot(h_ref[k], wh_ref[k],
                                preferred_element_type=jnp.float32)
        o_ref[0] = jnp.maximum(acc, 0.0)

    return pl.pallas_call(
        body,
        grid=(N // RB, 2),
        in_specs=[
            pl.BlockSpec((Ka, RB, D), lambda i, c: (0, i, 0)),
            pl.BlockSpec((Ka, D, D), lambda i, c: (0, 0, c)),
            pl.BlockSpec((Kh, RB, D), lambda i, c: (0, i, 0)),
            pl.BlockSpec((Kh, D, D), lambda i, c: (0, 0, c)),
            pl.BlockSpec((1, D), lambda i, c: (0, c)),
        ],
        out_specs=pl.BlockSpec((1, RB, D), lambda i, c: (c, i, 0)),
        out_shape=jax.ShapeDtypeStruct((2, N, D), jnp.float32),
    )


def _pool_mlp():
    nb = N // RB

    def body(b_ref, h_ref, w1_ref, b1_ref, w2t_ref, b2_ref, o_ref, acc, cnt):
        i = pl.program_id(0)

        @pl.when(i == 0)
        def _():
            acc[...] = jnp.zeros((G2, 2 * D), jnp.float32)
            cnt[...] = jnp.zeros((G2, D), jnp.float32)

        bcol = jnp.reshape(b_ref[0, 0, :], (RB, 1))
        P = (lax.broadcasted_iota(jnp.int32, (RB, G2), 1) == bcol)
        P = P.astype(jnp.float32)
        dn = (((0,), (0,)), ((), ()))
        acc[:, :D] = acc[:, :D] + lax.dot_general(
            P, h_ref[0], dn, preferred_element_type=jnp.float32)
        acc[:, D:] = acc[:, D:] + lax.dot_general(
            P, h_ref[1], dn, preferred_element_type=jnp.float32)
        cnt[...] = cnt[...] + lax.dot_general(
            P, jnp.ones((RB, D), jnp.float32), dn,
            preferred_element_type=jnp.float32)

        @pl.when(i == nb - 1)
        def _():
            c2 = jnp.concatenate([cnt[...], cnt[...]], axis=1)
            pooled = acc[...] / jnp.maximum(c2, 1.0)
            z = jnp.maximum(
                jnp.dot(pooled, w1_ref[...],
                        preferred_element_type=jnp.float32) + b1_ref[...],
                0.0)
            o = lax.dot_general(w2t_ref[...], z, (((1,), (1,)), ((), ())),
                                preferred_element_type=jnp.float32)
            o_ref[...] = o + b2_ref[...]

    return pl.pallas_call(
        body,
        grid=(nb,),
        in_specs=[
            pl.BlockSpec((1, 1, RB), lambda i: (i, 0, 0)),
            pl.BlockSpec((2, RB, D), lambda i: (0, i, 0)),
            pl.BlockSpec((2 * D, 2 * D), lambda i: (0, 0)),
            pl.BlockSpec((1, 2 * D), lambda i: (0, 0)),
            pl.BlockSpec((1, 2 * D), lambda i: (0, 0)),
            pl.BlockSpec((1, 1), lambda i: (0, 0)),
        ],
        out_specs=pl.BlockSpec((1, G2), lambda i: (0, 0)),
        out_shape=jax.ShapeDtypeStruct((1, G2), jnp.float32),
        scratch_shapes=[pltpu.VMEM((G2, 2 * D), jnp.float32),
                        pltpu.VMEM((G2, D), jnp.float32)],
    )


def kernel(x, edge_index, batch, Wrel0, Wroot0, b0, Wrel1, Wroot1, b1,
           Wrel2, Wroot2, b2, lin1_W, lin1_b, lin2_W, lin2_b):
    src = edge_index[0]
    dst = edge_index[1]
    npad = EPAD - E
    pidx = jnp.arange(npad, dtype=jnp.int32)
    srcp = jnp.concatenate([src, jnp.zeros((npad,), jnp.int32)])
    dstp = jnp.concatenate([dst, N + (pidx % 16)])

    si0 = srcp.reshape(NC, NS, CPT0, CH)
    di0 = dstp.reshape(NC, NS, CPT0, CH)
    si12 = jnp.stack([srcp, srcp + N]).reshape(NC, NS, CPT12, CH)
    di12 = jnp.broadcast_to(dstp, (NC, EPAD)).reshape(NC, NS, CPT12, CH)
    zr = jnp.zeros((128, D), jnp.float32)

    seg0 = _sc_segsum(CPT0)
    seg12 = _sc_segsum(CPT12)
    lay0 = _tc_layer(2, 1)
    lay = _tc_layer(2, 2)
    pool = _pool_mlp()

    a0 = seg0(x, si0, di0, zr)
    h1 = lay0(a0, jnp.broadcast_to(Wrel0, (2, D, H)), x[None],
              Wroot0[None], b0.reshape(1, H))
    a1 = seg12(h1.reshape(2 * N, D), si12, di12, zr)
    h2 = lay(a1, Wrel1.reshape(2, D, H), h1, Wroot1.reshape(2, D, H),
             b1.reshape(1, H))
    a2 = seg12(h2.reshape(2 * N, D), si12, di12, zr)
    h3 = lay(a2, Wrel2.reshape(2, D, H), h2, Wroot2.reshape(2, D, H),
             b2.reshape(1, H))
    out = pool(batch.reshape(N // RB, 1, RB), h3, lin1_W,
               lin1_b.reshape(1, H), lin2_W.reshape(1, H),
               lin2_b.reshape(1, 1))
    return out[0, :G]


# speedup vs baseline: 3.2372x; 1.2101x over previous
import functools

import jax
import jax.numpy as jnp
from jax import lax
from jax.experimental import pallas as pl
from jax.experimental.pallas import tpu as pltpu
from jax.experimental.pallas import tpu_sc as plsc

N, E, D, H, G = 10000, 320000, 128, 256, 64
NC, NS = 2, 16
CH = 128
IBL = 8
EPAD = 327680
CPT0 = EPAD // 2 // NS // CH
CPT12 = EPAD // NS // CH
OPT = 632
NOUT = NS * OPT
NACC = NOUT
ZPT = NACC // NS
RB = 1000
G2 = 128

_mesh = plsc.VectorSubcoreMesh(core_axis_name="c", subcore_axis_name="s")


def _sc_segsum(cpt):
    nb = cpt // IBL
    assert nb % 2 == 0 and IBL % 2 == 0

    @functools.partial(
        pl.kernel,
        mesh=_mesh,
        out_type=jax.ShapeDtypeStruct((NC, NOUT, D), jnp.float32),
        scratch_types=[
            pltpu.VMEM((IBL * CH,), jnp.int32),
            pltpu.VMEM((IBL * CH,), jnp.int32),
            pltpu.VMEM((IBL * CH,), jnp.int32),
            pltpu.VMEM((IBL * CH,), jnp.int32),
            pltpu.VMEM((CH, D), jnp.float32),
            pltpu.VMEM((CH, D), jnp.float32),
            pltpu.VMEM_SHARED((NACC, D), jnp.float32),
            pltpu.SemaphoreType.DMA,
            pltpu.SemaphoreType.DMA,
            pltpu.SemaphoreType.DMA,
            pltpu.SemaphoreType.DMA,
        ],
    )
    def seg(table, si, di, zr, out, sb0, sb1, db0, db1, rows0, rows1,
            acc, semr0, semr1, semi0, semi1):
        c = lax.axis_index("c")
        s = lax.axis_index("s")
        sb = (sb0, sb1)
        db = (db0, db1)
        rows = (rows0, rows1)
        semr = (semr0, semr1)
        semi = (semi0, semi1)

        pltpu.sync_copy(zr, rows0)
        base = s * ZPT
        for k in range(ZPT // 128):
            pltpu.sync_copy(rows0, acc.at[pl.ds(base + k * 128, 128)])
        rem = ZPT % 128
        pltpu.sync_copy(rows0.at[pl.ds(0, rem)],
                        acc.at[pl.ds(base + ZPT - rem, rem)])

        def load_idx(bi, p):
            sl = pl.ds(bi * IBL * CH, IBL * CH)
            pltpu.async_copy(si.at[c, s, sl], sb[p], semi[p])
            pltpu.async_copy(di.at[c, s, sl], db[p], semi[p])

        def wait_idx(p):
            sl = pl.ds(0, IBL * CH)
            pltpu.make_async_copy(si.at[c, s, sl], sb[p], semi[p]).wait()
            pltpu.make_async_copy(di.at[c, s, sl], db[p], semi[p]).wait()

        def fire(p, k, b):
            pltpu.async_copy(table.at[sb[p].at[pl.ds(k * CH, CH)]],
                             rows[b], semr[b])

        def drainr(b):
            pltpu.make_async_copy(table.at[pl.ds(0, CH)], rows[b],
                                  semr[b]).wait()

        def scat(p, k, b):
            pltpu.sync_copy(rows[b], acc.at[db[p].at[pl.ds(k * CH, CH)]],
                            add=True)

        def run_block(p, fire_next):
            for k in range(IBL - 2):
                drainr(k % 2)
                scat(p, k, k % 2)
                fire(p, k + 2, k % 2)
            if fire_next:
                wait_idx(1 - p)
            drainr(0)
            scat(p, IBL - 2, 0)
            if fire_next:
                fire(1 - p, 0, 0)
            drainr(1)
            scat(p, IBL - 1, 1)
            if fire_next:
                fire(1 - p, 1, 1)

        load_idx(0, 0)
        load_idx(1, 1)
        plsc.subcore_barrier()
        wait_idx(0)
        fire(0, 0, 0)
        fire(0, 1, 1)

        def pair(g, carry):
            run_block(0, True)
            load_idx(2 * g + 2, 0)
            run_block(1, True)
            load_idx(2 * g + 3, 1)
            return carry

        lax.fori_loop(0, nb // 2 - 1, pair, 0)
        run_block(0, True)
        run_block(1, False)
        plsc.subcore_barrier()

        pltpu.sync_copy(acc.at[pl.ds(s * OPT, OPT)],
                        out.at[c, pl.ds(s * OPT, OPT)])

    return seg


def _tc_layer(Ka, Kh):
    def body(a_ref, wa_ref, h_ref, wh_ref, b_ref, o_ref):
        acc = jnp.zeros((RB, D), jnp.float32) + b_ref[...]
        for k in range(Ka):
            acc = acc + jnp.dot(a_ref[k], wa_ref[k],
                                preferred_element_type=jnp.float32)
        for k in range(Kh):
            acc = acc + jnp.dot(h_ref[k], wh_ref[k],
                                preferred_element_type=jnp.float32)
        o_ref[0] = jnp.maximum(acc, 0.0)

    return pl.pallas_call(
        body,
        grid=(N // RB, 2),
        in_specs=[
            pl.BlockSpec((Ka, RB, D), lambda i, c: (0, i, 0)),
            pl.BlockSpec((Ka, D, D), lambda i, c: (0, 0, c)),
            pl.BlockSpec((Kh, RB, D), lambda i, c: (0, i, 0)),
            pl.BlockSpec((Kh, D, D), lambda i, c: (0, 0, c)),
            pl.BlockSpec((1, D), lambda i, c: (0, c)),
        ],
        out_specs=pl.BlockSpec((1, RB, D), lambda i, c: (c, i, 0)),
        out_shape=jax.ShapeDtypeStruct((2, N, D), jnp.float32),
    )


def _pool_mlp():
    nb = N // RB

    def body(b_ref, h_ref, w1_ref, b1_ref, w2t_ref, b2_ref, o_ref, acc, cnt):
        i = pl.program_id(0)

        @pl.when(i == 0)
        def _():
            acc[...] = jnp.zeros((G2, 2 * D), jnp.float32)
            cnt[...] = jnp.zeros((G2, D), jnp.float32)

        bcol = jnp.reshape(b_ref[0, 0, :], (RB, 1))
        P = (lax.broadcasted_iota(jnp.int32, (RB, G2), 1) == bcol)
        P = P.astype(jnp.float32)
        dn = (((0,), (0,)), ((), ()))
        acc[:, :D] = acc[:, :D] + lax.dot_general(
            P, h_ref[0], dn, preferred_element_type=jnp.float32)
        acc[:, D:] = acc[:, D:] + lax.dot_general(
            P, h_ref[1], dn, preferred_element_type=jnp.float32)
        cnt[...] = cnt[...] + lax.dot_general(
            P, jnp.ones((RB, D), jnp.float32), dn,
            preferred_element_type=jnp.float32)

        @pl.when(i == nb - 1)
        def _():
            c2 = jnp.concatenate([cnt[...], cnt[...]], axis=1)
            pooled = acc[...] / jnp.maximum(c2, 1.0)
            z = jnp.maximum(
                jnp.dot(pooled, w1_ref[...],
                        preferred_element_type=jnp.float32) + b1_ref[...],
                0.0)
            o = lax.dot_general(w2t_ref[...], z, (((1,), (1,)), ((), ())),
                                preferred_element_type=jnp.float32)
            o_ref[...] = o + b2_ref[...]

    return pl.pallas_call(
        body,
        grid=(nb,),
        in_specs=[
            pl.BlockSpec((1, 1, RB), lambda i: (i, 0, 0)),
            pl.BlockSpec((2, RB, D), lambda i: (0, i, 0)),
            pl.BlockSpec((2 * D, 2 * D), lambda i: (0, 0)),
            pl.BlockSpec((1, 2 * D), lambda i: (0, 0)),
            pl.BlockSpec((1, 2 * D), lambda i: (0, 0)),
            pl.BlockSpec((1, 1), lambda i: (0, 0)),
        ],
        out_specs=pl.BlockSpec((1, G2), lambda i: (0, 0)),
        out_shape=jax.ShapeDtypeStruct((1, G2), jnp.float32),
        scratch_shapes=[pltpu.VMEM((G2, 2 * D), jnp.float32),
                        pltpu.VMEM((G2, D), jnp.float32)],
    )


def kernel(x, edge_index, batch, Wrel0, Wroot0, b0, Wrel1, Wroot1, b1,
           Wrel2, Wroot2, b2, lin1_W, lin1_b, lin2_W, lin2_b):
    src = edge_index[0]
    dst = edge_index[1]
    npad = EPAD - E
    pidx = jnp.arange(npad, dtype=jnp.int32)
    srcp = jnp.concatenate([src, jnp.zeros((npad,), jnp.int32)])
    dstp = jnp.concatenate([dst, N + (pidx % 16)])

    si0 = srcp.reshape(NC, NS, CPT0 * CH)
    di0 = dstp.reshape(NC, NS, CPT0 * CH)
    si12 = jnp.stack([srcp, srcp + N]).reshape(NC, NS, CPT12 * CH)
    di12 = jnp.broadcast_to(dstp, (NC, EPAD)).reshape(NC, NS, CPT12 * CH)
    zr = jnp.zeros((128, D), jnp.float32)

    seg0 = _sc_segsum(CPT0)
    seg12 = _sc_segsum(CPT12)
    lay0 = _tc_layer(2, 1)
    lay = _tc_layer(2, 2)
    pool = _pool_mlp()

    a0 = seg0(x, si0, di0, zr)
    h1 = lay0(a0, jnp.broadcast_to(Wrel0, (2, D, H)), x[None],
              Wroot0[None], b0.reshape(1, H))
    a1 = seg12(h1.reshape(2 * N, D), si12, di12, zr)
    h2 = lay(a1, Wrel1.reshape(2, D, H), h1, Wroot1.reshape(2, D, H),
             b1.reshape(1, H))
    a2 = seg12(h2.reshape(2 * N, D), si12, di12, zr)
    h3 = lay(a2, Wrel2.reshape(2, D, H), h2, Wroot2.reshape(2, D, H),
             b2.reshape(1, H))
    out = pool(batch.reshape(N // RB, 1, RB), h3, lin1_W,
               lin1_b.reshape(1, H), lin2_W.reshape(1, H),
               lin2_b.reshape(1, 1))
    return out[0, :G]
